# baseline (speedup 1.0000x reference)
import functools

import jax
import jax.numpy as jnp
from jax import lax
from jax.experimental import pallas as pl
from jax.experimental.pallas import tpu as pltpu
from jax.experimental.pallas import tpu_sc as plsc

N = 10000
E = 320000
NT = 16
NG = 64
D = 128
NC = 2
NS = 16
NW = NC * NS
EPT = E // NW
CH = 640
NCH = 16
EPTP = NCH * CH
KG = 4
NGRP = NCH // KG
CSL = 10240
CPAD = NS * CSL
NRND = 4
GROWS = 2560
GH = GROWS * NG
GHSL = GH // NS
DUMP = GH
GHPAD = GH + 2048
GZSL = GHPAD // NS
GPAD = NRND * GH

_mesh = plsc.VectorSubcoreMesh(core_axis_name="c", subcore_axis_name="s")
_params = pltpu.CompilerParams(needs_layout_passes=False)


def _iota16():
    return lax.broadcasted_iota(jnp.int32, (16,), 0)


def _zero_fill(ref, n):
    def zchunk(i, _):
        ref[pl.ds(i * 16, 16)] = jnp.zeros((16,), jnp.float32)
        return ()
    lax.fori_loop(0, n // 16, zchunk, (), unroll=False)


@functools.partial(
    pl.kernel,
    out_type=jax.ShapeDtypeStruct((NC, 1, N), jnp.float32),
    mesh=_mesh,
    compiler_params=_params,
    scratch_types=[
        pltpu.VMEM((EPTP,), jnp.int32),
        pltpu.VMEM((NCH, 1, CH), jnp.int32),
        pltpu.VMEM((NCH, 1, CH), jnp.float32),
        pltpu.VMEM((N,), jnp.float32),
        pltpu.VMEM_SHARED((N,), jnp.float32),
        pltpu.SemaphoreType.DMA,
    ],
)
def _sc_deg(dst_hbm, deg_out, dstv, idx3, val3, zbuf, deg_sh, sem):
    c = lax.axis_index("c")
    s = lax.axis_index("s")
    row = c * NS + s
    pltpu.sync_copy(dst_hbm.at[row, 0], dstv)

    @pl.when(s == 0)
    def _():
        _zero_fill(zbuf, N)
        pltpu.sync_copy(zbuf, deg_sh)
    plsc.subcore_barrier()

    def group(g, _):
        base = g * KG
        for kk in range(KG):
            j = base + kk
            for k in range(CH // 16):
                o = j * CH + k * 16
                d16 = dstv[pl.ds(o, 16)]
                valid = (o + _iota16()) < EPT
                idx3[j, 0, pl.ds(k * 16, 16)] = d16
                val3[j, 0, pl.ds(k * 16, 16)] = jnp.where(valid, 1.0, 0.0)
            pltpu.async_copy(val3.at[j, 0], deg_sh.at[idx3.at[j, 0]], sem,
                             add=True)
        @pl.when(g > 0)
        def _():
            for kk in range(KG):
                jp = base - KG + kk
                pltpu.make_async_copy(
                    val3.at[jp, 0], deg_sh.at[idx3.at[jp, 0]], sem).wait()
        return ()

    lax.fori_loop(0, NGRP, group, (), unroll=False)
    for kk in range(KG):
        jp = (NGRP - 1) * KG + kk
        pltpu.make_async_copy(
            val3.at[jp, 0], deg_sh.at[idx3.at[jp, 0]], sem).wait()
    plsc.subcore_barrier()

    @pl.when(s == 0)
    def _():
        pltpu.sync_copy(deg_sh, zbuf)
        pltpu.sync_copy(zbuf, deg_out.at[c, 0])


@functools.partial(
    pl.kernel,
    out_type=(
        jax.ShapeDtypeStruct((NC, 1, CPAD), jnp.float32),
        jax.ShapeDtypeStruct((NC, 1, GPAD), jnp.float32),
    ),
    mesh=_mesh,
    compiler_params=_params,
    scratch_types=[
        pltpu.VMEM((EPTP,), jnp.int32),
        pltpu.VMEM((EPTP,), jnp.int32),
        pltpu.VMEM((N,), jnp.int32),
        pltpu.VMEM((N,), jnp.int32),
        pltpu.VMEM((N,), jnp.float32),
        pltpu.VMEM((NCH, 1, CH), jnp.int32),
        pltpu.VMEM((NCH, 1, CH), jnp.int32),
        pltpu.VMEM((NCH, 1, CH), jnp.float32),
        pltpu.VMEM((EPTP,), jnp.int32),
        pltpu.VMEM((GZSL,), jnp.float32),
        pltpu.VMEM_SHARED((CPAD,), jnp.float32),
        pltpu.VMEM_SHARED((GHPAD,), jnp.float32),
        pltpu.SemaphoreType.DMA,
    ],
)
def _sc_edges(src_hbm, dst_hbm, nt_hbm, ba_hbm, dv_hbm,
              c_out, g_out,
              srcv, dstv, ntv, bav, dvv, cidx, gidx, valv, bbuf,
              zbuf, c_sh, g_sh, sem):
    c = lax.axis_index("c")
    s = lax.axis_index("s")
    row = c * NS + s
    pltpu.sync_copy(src_hbm.at[row, 0], srcv)
    pltpu.sync_copy(dst_hbm.at[row, 0], dstv)
    pltpu.sync_copy(nt_hbm, ntv)
    pltpu.sync_copy(ba_hbm, bav)
    pltpu.sync_copy(dv_hbm, dvv)

    _zero_fill(zbuf, GZSL)
    pltpu.sync_copy(zbuf.at[pl.ds(0, CSL)], c_sh.at[pl.ds(s * CSL, CSL)])
    pltpu.sync_copy(zbuf, g_sh.at[pl.ds(s * GZSL, GZSL)])
    plsc.subcore_barrier()

    def group0(g, _):
        base = g * KG
        for kk in range(KG):
            j = base + kk
            for k in range(CH // 16):
                o = j * CH + k * 16
                s16 = srcv[pl.ds(o, 16)]
                d16 = dstv[pl.ds(o, 16)]
                t16 = plsc.load_gather(ntv, [s16])
                b16 = plsc.load_gather(bav, [d16])
                dvs = plsc.load_gather(dvv, [s16])
                dvd = plsc.load_gather(dvv, [d16])
                valid = (o + _iota16()) < EPT
                nrm = jnp.where(valid, dvs * dvd, 0.0)
                cidx[j, 0, pl.ds(k * 16, 16)] = d16 * NT + t16
                gidx[j, 0, pl.ds(k * 16, 16)] = jnp.where(
                    s16 < GROWS, s16 * NG + b16, DUMP + (d16 & 2047))
                valv[j, 0, pl.ds(k * 16, 16)] = nrm
                bbuf[pl.ds(o, 16)] = b16
            pltpu.async_copy(valv.at[j, 0], c_sh.at[cidx.at[j, 0]], sem,
                             add=True)
            pltpu.async_copy(valv.at[j, 0], g_sh.at[gidx.at[j, 0]], sem,
                             add=True)

        @pl.when(g > 0)
        def _():
            for kk in range(KG):
                jp = base - KG + kk
                pltpu.make_async_copy(
                    valv.at[jp, 0], c_sh.at[cidx.at[jp, 0]], sem).wait()
                pltpu.make_async_copy(
                    valv.at[jp, 0], g_sh.at[gidx.at[jp, 0]], sem).wait()
        return ()

    lax.fori_loop(0, NGRP, group0, (), unroll=False)
    for kk in range(KG):
        jp = (NGRP - 1) * KG + kk
        pltpu.make_async_copy(
            valv.at[jp, 0], c_sh.at[cidx.at[jp, 0]], sem).wait()
        pltpu.make_async_copy(
            valv.at[jp, 0], g_sh.at[gidx.at[jp, 0]], sem).wait()
    plsc.subcore_barrier()
    pltpu.sync_copy(c_sh.at[pl.ds(s * CSL, CSL)], zbuf.at[pl.ds(0, CSL)])
    pltpu.sync_copy(zbuf.at[pl.ds(0, CSL)], c_out.at[c, 0, pl.ds(s * CSL, CSL)])
    pltpu.sync_copy(g_sh.at[pl.ds(s * GHSL, GHSL)], zbuf.at[pl.ds(0, GHSL)])
    pltpu.sync_copy(zbuf.at[pl.ds(0, GHSL)],
                    g_out.at[c, 0, pl.ds(s * GHSL, GHSL)])
    plsc.subcore_barrier()

    for r in range(1, NRND):
        lo = r * GROWS
        _zero_fill(zbuf, GZSL)
        pltpu.sync_copy(zbuf, g_sh.at[pl.ds(s * GZSL, GZSL)])
        plsc.subcore_barrier()

        def group1(g, _):
            base = g * KG
            for kk in range(KG):
                j = base + kk
                for k in range(CH // 16):
                    o = j * CH + k * 16
                    s16 = srcv[pl.ds(o, 16)]
                    b16 = bbuf[pl.ds(o, 16)]
                    rel = s16 - lo
                    gidx[j, 0, pl.ds(k * 16, 16)] = jnp.where(
                        (rel >= 0) & (rel < GROWS), rel * NG + b16,
                        DUMP + (s16 & 2047))
                pltpu.async_copy(valv.at[j, 0], g_sh.at[gidx.at[j, 0]], sem,
                                 add=True)

            @pl.when(g > 0)
            def _():
                for kk in range(KG):
                    jp = base - KG + kk
                    pltpu.make_async_copy(
                        valv.at[jp, 0], g_sh.at[gidx.at[jp, 0]], sem).wait()
            return ()

        lax.fori_loop(0, NGRP, group1, (), unroll=False)
        for kk in range(KG):
            jp = (NGRP - 1) * KG + kk
            pltpu.make_async_copy(
                valv.at[jp, 0], g_sh.at[gidx.at[jp, 0]], sem).wait()
        plsc.subcore_barrier()
        pltpu.sync_copy(g_sh.at[pl.ds(s * GHSL, GHSL)],
                        zbuf.at[pl.ds(0, GHSL)])
        pltpu.sync_copy(zbuf.at[pl.ds(0, GHSL)],
                        g_out.at[c, 0, pl.ds(r * GH + s * GHSL, GHSL)])


def _tc1_body(deg_ref, emb_ref, w1_ref, dinv_ref, t1_ref):
    deg = deg_ref[0:1, :] + deg_ref[1:2, :] + 1.0
    dinv_ref[...] = lax.rsqrt(deg)
    t1_ref[...] = jnp.dot(emb_ref[...], w1_ref[...],
                          preferred_element_type=jnp.float32)


NB = 1000


def _tc2_body(c_ref, gt_ref, nt_ref, ba_ref, dv_ref, t1_ref, b1_ref,
              w2_ref, b2_ref, out_ref, acc, cnt):
    i = pl.program_id(0)

    @pl.when(i == 0)
    def _():
        acc[...] = jnp.zeros_like(acc)
        cnt[...] = jnp.zeros_like(cnt)

    dv = dv_ref[...]
    dv2 = dv * dv
    oh_t = (nt_ref[...] == lax.broadcasted_iota(jnp.int32, (1, NT), 1)
            ).astype(jnp.float32)
    cb = c_ref[...]
    cmat = cb[0] + cb[1] + oh_t * dv2
    x1 = jnp.maximum(
        jnp.dot(cmat, t1_ref[...], preferred_element_type=jnp.float32)
        + b1_ref[...], 0.0)
    h2 = jnp.dot(x1, w2_ref[...], preferred_element_type=jnp.float32)
    oh_g = (ba_ref[...] == lax.broadcasted_iota(jnp.int32, (1, NG), 1)
            ).astype(jnp.float32)
    gb = gt_ref[...]
    gmat = gb[0] + gb[1] + oh_g * dv2
    acc[...] += lax.dot_general(gmat, h2, (((0,), (0,)), ((), ())),
                                preferred_element_type=jnp.float32)
    cnt[...] += lax.dot_general(oh_g, jnp.ones((NB, 1), jnp.float32),
                                (((0,), (0,)), ((), ())),
                                preferred_element_type=jnp.float32)

    @pl.when(i == pl.num_programs(0) - 1)
    def _():
        cc = cnt[...]
        out_ref[...] = (acc[...] / jnp.maximum(cc, 1.0)
                        + b2_ref[...] * (cc > 0.0).astype(jnp.float32))


def kernel(node_type, edge_index, batch, embedding_table, W1, b1, W2, b2):
    src = edge_index[0].astype(jnp.int32)
    dst = edge_index[1].astype(jnp.int32)
    nt = node_type.astype(jnp.int32)
    ba = batch.astype(jnp.int32)
    srcp = jnp.pad(src.reshape(NW, 1, EPT), ((0, 0), (0, 0), (0, EPTP - EPT)))
    dstp = jnp.pad(dst.reshape(NW, 1, EPT), ((0, 0), (0, 0), (0, EPTP - EPT)))

    deg_part = _sc_deg(dstp)

    dinv2d, t1 = pl.pallas_call(
        _tc1_body,
        out_shape=(
            jax.ShapeDtypeStruct((1, N), jnp.float32),
            jax.ShapeDtypeStruct((NT, D), jnp.float32),
        ),
    )(deg_part.reshape(NC, N), embedding_table, W1)

    c_part, g_part = _sc_edges(srcp, dstp, nt, ba, dinv2d.reshape(N))

    out = pl.pallas_call(
        _tc2_body,
        grid=(N // NB,),
        in_specs=[
            pl.BlockSpec((NC, NB, NT), lambda i: (0, i, 0)),
            pl.BlockSpec((NC, NB, NG), lambda i: (0, i, 0)),
            pl.BlockSpec((NB, 1), lambda i: (i, 0)),
            pl.BlockSpec((NB, 1), lambda i: (i, 0)),
            pl.BlockSpec((NB, 1), lambda i: (i, 0)),
            pl.BlockSpec((NT, D), lambda i: (0, 0)),
            pl.BlockSpec((1, D), lambda i: (0, 0)),
            pl.BlockSpec((D, D), lambda i: (0, 0)),
            pl.BlockSpec((1, D), lambda i: (0, 0)),
        ],
        out_specs=pl.BlockSpec((NG, D), lambda i: (0, 0)),
        out_shape=jax.ShapeDtypeStruct((NG, D), jnp.float32),
        scratch_shapes=[
            pltpu.VMEM((NG, D), jnp.float32),
            pltpu.VMEM((NG, 1), jnp.float32),
        ],
    )(
        c_part.reshape(NC, CPAD // NT, NT),
        g_part.reshape(NC, GPAD // NG, NG),
        nt.reshape(N, 1),
        ba.reshape(N, 1),
        dinv2d.reshape(N, 1),
        t1,
        b1.reshape(1, D),
        W2,
        b2.reshape(1, D),
    )
    return out


# speedup vs baseline: 1.0091x; 1.0091x over previous
import functools

import jax
import jax.numpy as jnp
from jax import lax
from jax.experimental import pallas as pl
from jax.experimental.pallas import tpu as pltpu
from jax.experimental.pallas import tpu_sc as plsc

N = 10000
E = 320000
NT = 16
NG = 64
D = 128
NC = 2
NS = 16
NW = NC * NS
EPT = E // NW
CH = 512
NCH = 20
EPTP = NCH * CH
KG = 4
NGRP = NCH // KG
CSL = 10240
CPAD = NS * CSL
NRND = 4
GROWS = 2560
GH = GROWS * NG
GHSL = GH // NS
DUMP = GH
GHPAD = GH + 2048
GZSL = GHPAD // NS
GPAD = NRND * GH

_mesh = plsc.VectorSubcoreMesh(core_axis_name="c", subcore_axis_name="s")
_params = pltpu.CompilerParams(needs_layout_passes=False)


def _iota16():
    return lax.broadcasted_iota(jnp.int32, (16,), 0)


def _zero_fill(ref, n):
    def zchunk(i, _):
        ref[pl.ds(i * 16, 16)] = jnp.zeros((16,), jnp.float32)
        return ()
    lax.fori_loop(0, n // 16, zchunk, (), unroll=False)


@functools.partial(
    pl.kernel,
    out_type=jax.ShapeDtypeStruct((NC, 1, N), jnp.float32),
    mesh=_mesh,
    compiler_params=_params,
    scratch_types=[
        pltpu.VMEM((EPTP,), jnp.int32),
        pltpu.VMEM((NCH, 1, CH), jnp.int32),
        pltpu.VMEM((NCH, 1, CH), jnp.float32),
        pltpu.VMEM((N,), jnp.float32),
        pltpu.VMEM_SHARED((N,), jnp.float32),
        pltpu.SemaphoreType.DMA,
    ],
)
def _sc_deg(dst_hbm, deg_out, dstv, idx3, val3, zbuf, deg_sh, sem):
    c = lax.axis_index("c")
    s = lax.axis_index("s")
    row = c * NS + s
    pltpu.sync_copy(dst_hbm.at[row, 0], dstv)

    @pl.when(s == 0)
    def _():
        _zero_fill(zbuf, N)
        pltpu.sync_copy(zbuf, deg_sh)
    plsc.subcore_barrier()

    def group(g, _):
        base = g * KG
        for kk in range(KG):
            j = base + kk
            for k in range(CH // 16):
                o = j * CH + k * 16
                d16 = dstv[pl.ds(o, 16)]
                valid = (o + _iota16()) < EPT
                idx3[j, 0, pl.ds(k * 16, 16)] = d16
                val3[j, 0, pl.ds(k * 16, 16)] = jnp.where(valid, 1.0, 0.0)
            pltpu.async_copy(val3.at[j, 0], deg_sh.at[idx3.at[j, 0]], sem,
                             add=True)
        @pl.when(g > 0)
        def _():
            for kk in range(KG):
                jp = base - KG + kk
                pltpu.make_async_copy(
                    val3.at[jp, 0], deg_sh.at[idx3.at[jp, 0]], sem).wait()
        return ()

    lax.fori_loop(0, NGRP, group, (), unroll=False)
    for kk in range(KG):
        jp = (NGRP - 1) * KG + kk
        pltpu.make_async_copy(
            val3.at[jp, 0], deg_sh.at[idx3.at[jp, 0]], sem).wait()
    plsc.subcore_barrier()

    @pl.when(s == 0)
    def _():
        pltpu.sync_copy(deg_sh, zbuf)
        pltpu.sync_copy(zbuf, deg_out.at[c, 0])


@functools.partial(
    pl.kernel,
    out_type=(
        jax.ShapeDtypeStruct((NC, 1, CPAD), jnp.float32),
        jax.ShapeDtypeStruct((NC, 1, GPAD), jnp.float32),
    ),
    mesh=_mesh,
    compiler_params=_params,
    scratch_types=[
        pltpu.VMEM((EPTP,), jnp.int32),
        pltpu.VMEM((EPTP,), jnp.int32),
        pltpu.VMEM((N,), jnp.int32),
        pltpu.VMEM((N,), jnp.int32),
        pltpu.VMEM((N,), jnp.float32),
        pltpu.VMEM((NCH, 1, CH), jnp.int32),
        pltpu.VMEM((NCH, 1, CH), jnp.int32),
        pltpu.VMEM((NCH, 1, CH), jnp.float32),
        pltpu.VMEM((EPTP,), jnp.int32),
        pltpu.VMEM((GZSL,), jnp.float32),
        pltpu.VMEM_SHARED((CPAD,), jnp.float32),
        pltpu.VMEM_SHARED((GHPAD,), jnp.float32),
        pltpu.SemaphoreType.DMA,
    ],
)
def _sc_edges(src_hbm, dst_hbm, nt_hbm, ba_hbm, dv_hbm,
              c_out, g_out,
              srcv, dstv, ntv, bav, dvv, cidx, gidx, valv, bbuf,
              zbuf, c_sh, g_sh, sem):
    c = lax.axis_index("c")
    s = lax.axis_index("s")
    row = c * NS + s
    pltpu.sync_copy(src_hbm.at[row, 0], srcv)
    pltpu.sync_copy(dst_hbm.at[row, 0], dstv)
    pltpu.sync_copy(nt_hbm, ntv)
    pltpu.sync_copy(ba_hbm, bav)
    pltpu.sync_copy(dv_hbm, dvv)

    _zero_fill(zbuf, GZSL)
    pltpu.sync_copy(zbuf.at[pl.ds(0, CSL)], c_sh.at[pl.ds(s * CSL, CSL)])
    pltpu.sync_copy(zbuf, g_sh.at[pl.ds(s * GZSL, GZSL)])
    plsc.subcore_barrier()

    def group0(g, _):
        base = g * KG
        for kk in range(KG):
            j = base + kk
            for k in range(CH // 16):
                o = j * CH + k * 16
                s16 = srcv[pl.ds(o, 16)]
                d16 = dstv[pl.ds(o, 16)]
                t16 = plsc.load_gather(ntv, [s16])
                b16 = plsc.load_gather(bav, [d16])
                dvs = plsc.load_gather(dvv, [s16])
                dvd = plsc.load_gather(dvv, [d16])
                valid = (o + _iota16()) < EPT
                nrm = jnp.where(valid, dvs * dvd, 0.0)
                cidx[j, 0, pl.ds(k * 16, 16)] = d16 * NT + t16
                gidx[j, 0, pl.ds(k * 16, 16)] = jnp.where(
                    s16 < GROWS, s16 * NG + b16, DUMP + (d16 & 2047))
                valv[j, 0, pl.ds(k * 16, 16)] = nrm
                bbuf[pl.ds(o, 16)] = b16
            pltpu.async_copy(valv.at[j, 0], c_sh.at[cidx.at[j, 0]], sem,
                             add=True)
            pltpu.async_copy(valv.at[j, 0], g_sh.at[gidx.at[j, 0]], sem,
                             add=True)

        @pl.when(g > 0)
        def _():
            for kk in range(KG):
                jp = base - KG + kk
                pltpu.make_async_copy(
                    valv.at[jp, 0], c_sh.at[cidx.at[jp, 0]], sem).wait()
                pltpu.make_async_copy(
                    valv.at[jp, 0], g_sh.at[gidx.at[jp, 0]], sem).wait()
        return ()

    lax.fori_loop(0, NGRP, group0, (), unroll=False)
    for kk in range(KG):
        jp = (NGRP - 1) * KG + kk
        pltpu.make_async_copy(
            valv.at[jp, 0], c_sh.at[cidx.at[jp, 0]], sem).wait()
        pltpu.make_async_copy(
            valv.at[jp, 0], g_sh.at[gidx.at[jp, 0]], sem).wait()
    plsc.subcore_barrier()
    pltpu.sync_copy(c_sh.at[pl.ds(s * CSL, CSL)], zbuf.at[pl.ds(0, CSL)])
    pltpu.sync_copy(zbuf.at[pl.ds(0, CSL)], c_out.at[c, 0, pl.ds(s * CSL, CSL)])
    pltpu.sync_copy(g_sh.at[pl.ds(s * GHSL, GHSL)], zbuf.at[pl.ds(0, GHSL)])
    pltpu.sync_copy(zbuf.at[pl.ds(0, GHSL)],
                    g_out.at[c, 0, pl.ds(s * GHSL, GHSL)])
    plsc.subcore_barrier()

    for r in range(1, NRND):
        lo = r * GROWS
        _zero_fill(zbuf, GZSL)
        pltpu.sync_copy(zbuf, g_sh.at[pl.ds(s * GZSL, GZSL)])
        plsc.subcore_barrier()

        def group1(g, _):
            base = g * KG
            for kk in range(KG):
                j = base + kk
                for k in range(CH // 16):
                    o = j * CH + k * 16
                    s16 = srcv[pl.ds(o, 16)]
                    b16 = bbuf[pl.ds(o, 16)]
                    rel = s16 - lo
                    gidx[j, 0, pl.ds(k * 16, 16)] = jnp.where(
                        (rel >= 0) & (rel < GROWS), rel * NG + b16,
                        DUMP + (s16 & 2047))
                pltpu.async_copy(valv.at[j, 0], g_sh.at[gidx.at[j, 0]], sem,
                                 add=True)

            @pl.when(g > 0)
            def _():
                for kk in range(KG):
                    jp = base - KG + kk
                    pltpu.make_async_copy(
                        valv.at[jp, 0], g_sh.at[gidx.at[jp, 0]], sem).wait()
            return ()

        lax.fori_loop(0, NGRP, group1, (), unroll=False)
        for kk in range(KG):
            jp = (NGRP - 1) * KG + kk
            pltpu.make_async_copy(
                valv.at[jp, 0], g_sh.at[gidx.at[jp, 0]], sem).wait()
        plsc.subcore_barrier()
        pltpu.sync_copy(g_sh.at[pl.ds(s * GHSL, GHSL)],
                        zbuf.at[pl.ds(0, GHSL)])
        pltpu.sync_copy(zbuf.at[pl.ds(0, GHSL)],
                        g_out.at[c, 0, pl.ds(r * GH + s * GHSL, GHSL)])


def _tc1_body(deg_ref, emb_ref, w1_ref, dinv_ref, t1_ref):
    deg = deg_ref[0:1, :] + deg_ref[1:2, :] + 1.0
    dinv_ref[...] = lax.rsqrt(deg)
    t1_ref[...] = jnp.dot(emb_ref[...], w1_ref[...],
                          preferred_element_type=jnp.float32)


NB = 1000


def _tc2_body(c_ref, gt_ref, nt_ref, ba_ref, dv_ref, t1_ref, b1_ref,
              w2_ref, b2_ref, out_ref, acc, cnt):
    i = pl.program_id(0)

    @pl.when(i == 0)
    def _():
        acc[...] = jnp.zeros_like(acc)
        cnt[...] = jnp.zeros_like(cnt)

    dv = dv_ref[...]
    dv2 = dv * dv
    oh_t = (nt_ref[...] == lax.broadcasted_iota(jnp.int32, (1, NT), 1)
            ).astype(jnp.float32)
    cb = c_ref[...]
    cmat = cb[0] + cb[1] + oh_t * dv2
    x1 = jnp.maximum(
        jnp.dot(cmat, t1_ref[...], preferred_element_type=jnp.float32)
        + b1_ref[...], 0.0)
    h2 = jnp.dot(x1, w2_ref[...], preferred_element_type=jnp.float32)
    oh_g = (ba_ref[...] == lax.broadcasted_iota(jnp.int32, (1, NG), 1)
            ).astype(jnp.float32)
    gb = gt_ref[...]
    gmat = gb[0] + gb[1] + oh_g * dv2
    acc[...] += lax.dot_general(gmat, h2, (((0,), (0,)), ((), ())),
                                preferred_element_type=jnp.float32)
    cnt[...] += lax.dot_general(oh_g, jnp.ones((NB, 1), jnp.float32),
                                (((0,), (0,)), ((), ())),
                                preferred_element_type=jnp.float32)

    @pl.when(i == pl.num_programs(0) - 1)
    def _():
        cc = cnt[...]
        out_ref[...] = (acc[...] / jnp.maximum(cc, 1.0)
                        + b2_ref[...] * (cc > 0.0).astype(jnp.float32))


def kernel(node_type, edge_index, batch, embedding_table, W1, b1, W2, b2):
    src = edge_index[0].astype(jnp.int32)
    dst = edge_index[1].astype(jnp.int32)
    nt = node_type.astype(jnp.int32)
    ba = batch.astype(jnp.int32)
    srcp = jnp.pad(src.reshape(NW, 1, EPT), ((0, 0), (0, 0), (0, EPTP - EPT)))
    dstp = jnp.pad(dst.reshape(NW, 1, EPT), ((0, 0), (0, 0), (0, EPTP - EPT)))

    deg_part = _sc_deg(dstp)

    dinv2d, t1 = pl.pallas_call(
        _tc1_body,
        out_shape=(
            jax.ShapeDtypeStruct((1, N), jnp.float32),
            jax.ShapeDtypeStruct((NT, D), jnp.float32),
        ),
    )(deg_part.reshape(NC, N), embedding_table, W1)

    c_part, g_part = _sc_edges(srcp, dstp, nt, ba, dinv2d.reshape(N))

    out = pl.pallas_call(
        _tc2_body,
        grid=(N // NB,),
        in_specs=[
            pl.BlockSpec((NC, NB, NT), lambda i: (0, i, 0)),
            pl.BlockSpec((NC, NB, NG), lambda i: (0, i, 0)),
            pl.BlockSpec((NB, 1), lambda i: (i, 0)),
            pl.BlockSpec((NB, 1), lambda i: (i, 0)),
            pl.BlockSpec((NB, 1), lambda i: (i, 0)),
            pl.BlockSpec((NT, D), lambda i: (0, 0)),
            pl.BlockSpec((1, D), lambda i: (0, 0)),
            pl.BlockSpec((D, D), lambda i: (0, 0)),
            pl.BlockSpec((1, D), lambda i: (0, 0)),
        ],
        out_specs=pl.BlockSpec((NG, D), lambda i: (0, 0)),
        out_shape=jax.ShapeDtypeStruct((NG, D), jnp.float32),
        scratch_shapes=[
            pltpu.VMEM((NG, D), jnp.float32),
            pltpu.VMEM((NG, 1), jnp.float32),
        ],
    )(
        c_part.reshape(NC, CPAD // NT, NT),
        g_part.reshape(NC, GPAD // NG, NG),
        nt.reshape(N, 1),
        ba.reshape(N, 1),
        dinv2d.reshape(N, 1),
        t1,
        b1.reshape(1, D),
        W2,
        b2.reshape(1, D),
    )
    return out
